# baseline (speedup 1.0000x reference)
import jax
import jax.numpy as jnp
from jax import lax
from jax.experimental import pallas as pl
from jax.experimental.pallas import tpu as pltpu
from jax.experimental.pallas import tpu_sc as plsc

_NUM_EXPERTS = 16
_L = 16
_NS = 16
_NUM_LAYERS = 24
_TOKENS = 4 * 4096
_TOK_PER_TILE = _TOKENS // _NS
_CHUNKS = _TOK_PER_TILE // _L
_UNROLL = 8
_OUTER = _CHUNKS // _UNROLL

_TOTAL_CAPITAL = 10000.0
_MIN_CAP = _TOTAL_CAPITAL * 0.05 / _NUM_EXPERTS
_TAX_THRESHOLD = 2.0
_TAX_RATE = 0.1


def _sc_body(loss_h, cost_h, w_h, caps_h, base_h, li_h, caps_out, base_out,
             loss_v, cost_v, w_v, cnt_v, s_v, part_v,
             li_v, base24_v, caps24_v, shared, all_v):
    sid = lax.axis_index("s")
    off = sid * _TOK_PER_TILE
    pltpu.sync_copy(loss_h.at[pl.ds(off, _TOK_PER_TILE)], loss_v)
    pltpu.sync_copy(cost_h.at[pl.ds(off, _TOK_PER_TILE)], cost_v)
    pltpu.sync_copy(w_h.at[pl.ds(2 * off, 2 * _TOK_PER_TILE)], w_v)

    zeros = jnp.zeros((_L,), jnp.float32)
    ones = jnp.ones((_L,), jnp.float32)
    lanes = lax.iota(jnp.int32, _L)
    pair_tok = lax.shift_right_logical(lanes, 1)
    pair_base = lanes - (lanes & 1)
    even = (lanes & 1) == 0
    cnt_v[...] = zeros
    s_v[...] = zeros

    def outer(j, loss_acc):
        for k in range(_UNROLL):
            i = j * _UNROLL + k
            lo = loss_v[pl.ds(i * _L, _L)]
            lc = lo + cost_v[pl.ds(i * _L, _L)]
            loss_acc = loss_acc + lo
            for h in range(2):
                wv = w_v[pl.ds(i * 2 * _L + h * _L, _L)]
                wsh = jnp.take_along_axis(wv, pair_base, axis=0,
                                          mode="promise_in_bounds")
                m = even | (wv != wsh)
                lce = jnp.take_along_axis(lc, pair_tok + (8 * h), axis=0,
                                          mode="promise_in_bounds")
                plsc.addupdate_scatter(s_v, [wv], lce, mask=m)
                plsc.addupdate_scatter(cnt_v, [wv], ones, mask=m)
        return loss_acc

    loss_acc = lax.fori_loop(0, _OUTER, outer, zeros)

    part_v[pl.ds(0, _L)] = cnt_v[...]
    part_v[pl.ds(_L, _L)] = s_v[...]
    part_v[pl.ds(2 * _L, _L)] = loss_acc
    pltpu.sync_copy(part_v, shared.at[pl.ds(sid * 3 * _L, 3 * _L)])
    plsc.subcore_barrier()

    @pl.when(sid == 0)
    def _finalize():
        pltpu.sync_copy(shared, all_v)
        pltpu.sync_copy(li_h, li_v)
        pltpu.sync_copy(caps_h, caps24_v)
        pltpu.sync_copy(base_h, base24_v)
        li_vec = li_v[...]
        cnt = all_v[pl.ds(0, _L)]
        s = all_v[pl.ds(_L, _L)]
        lsum = all_v[pl.ds(2 * _L, _L)]
        for i in range(1, _NS):
            cnt = cnt + all_v[pl.ds(i * 3 * _L, _L)]
            s = s + all_v[pl.ds((i * 3 + 1) * _L, _L)]
            lsum = lsum + all_v[pl.ds((i * 3 + 2) * _L, _L)]
        avg_loss = jnp.sum(lsum) * (1.0 / _TOKENS)
        base_splat = plsc.load_gather(base24_v, [li_vec])
        new_base = 0.99 * base_splat + 0.01 * avg_loss
        crow = plsc.load_gather(caps24_v, [li_vec, lanes])
        caps = crow + new_base * cnt - s
        thr = jnp.sum(caps) * (_TAX_THRESHOLD / _NUM_EXPERTS)
        caps = jnp.where(caps > thr, caps - (caps - thr) * _TAX_RATE, caps)
        caps = jnp.maximum(caps, _MIN_CAP)
        total = jnp.sum(caps)
        scale = jnp.where(total > _TOTAL_CAPITAL * 1.5, 0.95, 1.0)
        shift = jnp.where(total < _TOTAL_CAPITAL * 0.5,
                          _TOTAL_CAPITAL * 0.01, 0.0)
        caps = caps * scale + shift
        plsc.store_scatter(caps24_v, [li_vec, lanes], caps)
        plsc.store_scatter(base24_v, [li_vec], new_base, mask=lanes == 0)
        pltpu.sync_copy(caps24_v, caps_out)
        pltpu.sync_copy(base24_v, base_out)


_mesh = plsc.VectorSubcoreMesh(
    core_axis_name="c", subcore_axis_name="s", num_cores=1, num_subcores=_NS)

_sc_call = pl.kernel(
    _sc_body,
    out_type=(
        jax.ShapeDtypeStruct((_NUM_LAYERS, _NUM_EXPERTS), jnp.float32),
        jax.ShapeDtypeStruct((_NUM_LAYERS,), jnp.float32),
    ),
    mesh=_mesh,
    scratch_types=[
        pltpu.VMEM((_TOK_PER_TILE,), jnp.float32),
        pltpu.VMEM((_TOK_PER_TILE,), jnp.float32),
        pltpu.VMEM((2 * _TOK_PER_TILE,), jnp.int32),
        pltpu.VMEM((_NUM_EXPERTS,), jnp.float32),
        pltpu.VMEM((_NUM_EXPERTS,), jnp.float32),
        pltpu.VMEM((3 * _L,), jnp.float32),
        pltpu.VMEM((_L,), jnp.int32),
        pltpu.VMEM((_NUM_LAYERS,), jnp.float32),
        pltpu.VMEM((_NUM_LAYERS, _NUM_EXPERTS), jnp.float32),
        pltpu.VMEM_SHARED((_NS * 3 * _L,), jnp.float32),
        pltpu.VMEM((_NS * 3 * _L,), jnp.float32),
    ],
    compiler_params=pltpu.CompilerParams(needs_layout_passes=False),
    name="capital_manager_sc",
)


def kernel(capitals, baseline_losses, token_losses, costs, winners, layer_idx):
    li_arr = jnp.full((_L,), layer_idx, dtype=jnp.int32)
    loss_flat = token_losses.reshape(_TOKENS)
    cost_flat = costs.reshape(_TOKENS)
    w_flat = winners.reshape(2 * _TOKENS)
    new_capitals, new_baselines = _sc_call(
        loss_flat, cost_flat, w_flat, capitals, baseline_losses, li_arr)
    return new_capitals, new_baselines


# speedup vs baseline: 1.4306x; 1.4306x over previous
import jax
import jax.numpy as jnp
from jax import lax
from jax.experimental import pallas as pl
from jax.experimental.pallas import tpu as pltpu
from jax.experimental.pallas import tpu_sc as plsc

_NUM_EXPERTS = 16
_L = 16
_NS = 16
_NUM_LAYERS = 24
_TOKENS = 4 * 4096
_TOK_PER_TILE = _TOKENS // _NS
_CHUNKS = _TOK_PER_TILE // _L
_BANKS = 4

_TOTAL_CAPITAL = 10000.0
_MIN_CAP = _TOTAL_CAPITAL * 0.05 / _NUM_EXPERTS
_TAX_THRESHOLD = 2.0
_TAX_RATE = 0.1


def _sc_body(loss_h, cost_h, w0_h, w1_h, caps_h, base_h, li_h,
             caps_out, base_out,
             loss_v, cost_v, w0_v, w1_v, bins_v, part_v,
             li_v, base24_v, caps24_v, shared, all_v, sem):
    sid = lax.axis_index("s")
    off = sid * _TOK_PER_TILE
    d1 = pltpu.async_copy(loss_h.at[pl.ds(off, _TOK_PER_TILE)], loss_v, sem)
    d2 = pltpu.async_copy(cost_h.at[pl.ds(off, _TOK_PER_TILE)], cost_v, sem)
    d3 = pltpu.async_copy(w0_h.at[pl.ds(off, _TOK_PER_TILE)], w0_v, sem)
    d4 = pltpu.async_copy(w1_h.at[pl.ds(off, _TOK_PER_TILE)], w1_v, sem)

    zeros = jnp.zeros((_L,), jnp.float32)
    ones = jnp.ones((_L,), jnp.float32)
    lanes = lax.iota(jnp.int32, _L)
    for b in range(2 * _BANKS):
        bins_v[pl.ds(b * _L, _L)] = zeros

    d1.wait()
    d2.wait()
    d3.wait()
    d4.wait()

    loss_accs = [zeros] * _BANKS
    for i in range(_CHUNKS):
        sl = pl.ds(i * _L, _L)
        lo = loss_v[sl]
        lc = lo + cost_v[sl]
        a = w0_v[sl]
        b = w1_v[sl]
        m = b != a
        bk0 = (2 * i) % _BANKS
        bk1 = (2 * i + 1) % _BANKS
        plsc.addupdate_scatter(bins_v.at[pl.ds(bk0 * 2 * _L, _L)], [a], ones)
        plsc.addupdate_scatter(bins_v.at[pl.ds(bk0 * 2 * _L + _L, _L)],
                               [a], lc)
        plsc.addupdate_scatter(bins_v.at[pl.ds(bk1 * 2 * _L, _L)], [b], ones,
                               mask=m)
        plsc.addupdate_scatter(bins_v.at[pl.ds(bk1 * 2 * _L + _L, _L)],
                               [b], lc, mask=m)
        loss_accs[i % _BANKS] = loss_accs[i % _BANKS] + lo

    cnt_p = zeros
    s_p = zeros
    for bnk in range(_BANKS):
        cnt_p = cnt_p + bins_v[pl.ds(bnk * 2 * _L, _L)]
        s_p = s_p + bins_v[pl.ds(bnk * 2 * _L + _L, _L)]
    loss_acc = (loss_accs[0] + loss_accs[1]) + (loss_accs[2] + loss_accs[3])

    part_v[pl.ds(0, _L)] = cnt_p
    part_v[pl.ds(_L, _L)] = s_p
    part_v[pl.ds(2 * _L, _L)] = loss_acc
    pltpu.sync_copy(part_v, shared.at[pl.ds(sid * 3 * _L, 3 * _L)])
    plsc.subcore_barrier()

    @pl.when(sid == 0)
    def _finalize():
        pltpu.sync_copy(shared, all_v)
        pltpu.sync_copy(li_h, li_v)
        pltpu.sync_copy(caps_h, caps24_v)
        pltpu.sync_copy(base_h, base24_v)
        li_vec = li_v[...]
        cnt = all_v[pl.ds(0, _L)]
        s = all_v[pl.ds(_L, _L)]
        lsum = all_v[pl.ds(2 * _L, _L)]
        for i in range(1, _NS):
            cnt = cnt + all_v[pl.ds(i * 3 * _L, _L)]
            s = s + all_v[pl.ds((i * 3 + 1) * _L, _L)]
            lsum = lsum + all_v[pl.ds((i * 3 + 2) * _L, _L)]
        avg_loss = jnp.sum(lsum) * (1.0 / _TOKENS)
        base_splat = plsc.load_gather(base24_v, [li_vec])
        new_base = 0.99 * base_splat + 0.01 * avg_loss
        crow = plsc.load_gather(caps24_v, [li_vec, lanes])
        caps = crow + new_base * cnt - s
        thr = jnp.sum(caps) * (_TAX_THRESHOLD / _NUM_EXPERTS)
        caps = jnp.where(caps > thr, caps - (caps - thr) * _TAX_RATE, caps)
        caps = jnp.maximum(caps, _MIN_CAP)
        total = jnp.sum(caps)
        scale = jnp.where(total > _TOTAL_CAPITAL * 1.5, 0.95, 1.0)
        shift = jnp.where(total < _TOTAL_CAPITAL * 0.5,
                          _TOTAL_CAPITAL * 0.01, 0.0)
        caps = caps * scale + shift
        plsc.store_scatter(caps24_v, [li_vec, lanes], caps)
        plsc.store_scatter(base24_v, [li_vec], new_base, mask=lanes == 0)
        pltpu.sync_copy(caps24_v, caps_out)
        pltpu.sync_copy(base24_v, base_out)


_mesh = plsc.VectorSubcoreMesh(
    core_axis_name="c", subcore_axis_name="s", num_cores=1, num_subcores=_NS)

_sc_call = pl.kernel(
    _sc_body,
    out_type=(
        jax.ShapeDtypeStruct((_NUM_LAYERS, _NUM_EXPERTS), jnp.float32),
        jax.ShapeDtypeStruct((_NUM_LAYERS,), jnp.float32),
    ),
    mesh=_mesh,
    scratch_types=[
        pltpu.VMEM((_TOK_PER_TILE,), jnp.float32),
        pltpu.VMEM((_TOK_PER_TILE,), jnp.float32),
        pltpu.VMEM((_TOK_PER_TILE,), jnp.int32),
        pltpu.VMEM((_TOK_PER_TILE,), jnp.int32),
        pltpu.VMEM((_BANKS * 2 * _L,), jnp.float32),
        pltpu.VMEM((3 * _L,), jnp.float32),
        pltpu.VMEM((_L,), jnp.int32),
        pltpu.VMEM((_NUM_LAYERS,), jnp.float32),
        pltpu.VMEM((_NUM_LAYERS, _NUM_EXPERTS), jnp.float32),
        pltpu.VMEM_SHARED((_NS * 3 * _L,), jnp.float32),
        pltpu.VMEM((_NS * 3 * _L,), jnp.float32),
        pltpu.SemaphoreType.DMA,
    ],
    compiler_params=pltpu.CompilerParams(needs_layout_passes=False),
    name="capital_manager_sc",
)


def kernel(capitals, baseline_losses, token_losses, costs, winners, layer_idx):
    li_arr = jnp.full((_L,), layer_idx, dtype=jnp.int32)
    loss_flat = token_losses.reshape(_TOKENS)
    cost_flat = costs.reshape(_TOKENS)
    w = winners.reshape(_TOKENS, 2)
    w0 = w[:, 0]
    w1 = w[:, 1]
    new_capitals, new_baselines = _sc_call(
        loss_flat, cost_flat, w0, w1, capitals, baseline_losses, li_arr)
    return new_capitals, new_baselines
